# baseline (speedup 1.0000x reference)
import functools

import jax
import jax.numpy as jnp
from jax import lax
from jax.experimental import pallas as pl
from jax.experimental.pallas import tpu as pltpu
from jax.experimental.pallas import tpu_sc as plsc

_NC = 2
_NS = 16
_NW = _NC * _NS
_CHUNK = 80
_NBUF = 2
_LOOKAHEAD = 1


@functools.partial(jax.jit, static_argnames=("dim",))
def _gather_rows(idx, weight, dim):
    n = idx.shape[0]
    per_w = n // _NW
    cpw = per_w // _CHUNK
    mesh = plsc.VectorSubcoreMesh(core_axis_name="c", subcore_axis_name="s")

    @functools.partial(
        pl.kernel,
        mesh=mesh,
        compiler_params=pltpu.CompilerParams(use_tc_tiling_on_sc=False),
        out_type=jax.ShapeDtypeStruct((n, dim), jnp.float32),
        scratch_types=[
            pltpu.VMEM((per_w,), jnp.int32),
            *[pltpu.VMEM((_CHUNK, dim), jnp.float32) for _ in range(_NBUF)],
            *[pltpu.SemaphoreType.DMA for _ in range(2 * _NBUF)],
        ],
    )
    def k(idx_hbm, table_hbm, out_hbm, idx_v, *bufs_sems):
        rows = bufs_sems[:_NBUF]
        gsem = bufs_sems[_NBUF:2 * _NBUF]
        osem = bufs_sems[2 * _NBUF:]
        wid = lax.axis_index("s") * _NC + lax.axis_index("c")
        base = wid * per_w
        pltpu.sync_copy(idx_hbm.at[pl.ds(base, per_w)], idx_v)

        def fire_gather(j):
            b = j % _NBUF
            return pltpu.async_copy(
                table_hbm.at[idx_v.at[pl.ds(j * _CHUNK, _CHUNK)]],
                rows[b], gsem[b])

        def fire_out(j):
            b = j % _NBUF
            return pltpu.async_copy(
                rows[b], out_hbm.at[pl.ds(base + j * _CHUNK, _CHUNK)],
                osem[b])

        g = {}
        o = {}
        for j in range(min(_LOOKAHEAD, cpw)):
            g[j] = fire_gather(j)
        for j in range(cpw):
            nj = j + _LOOKAHEAD
            if nj < cpw:
                if nj >= _NBUF:
                    o[nj - _NBUF].wait()
                g[nj] = fire_gather(nj)
            g[j].wait()
            o[j] = fire_out(j)
        for j in range(max(0, cpw - _NBUF), cpw):
            o[j].wait()

    return k(idx, weight)


def kernel(tensor, weight):
    b, h = tensor.shape
    dim = weight.shape[1]
    idx = tensor.reshape(-1).astype(jnp.int32)
    out = _gather_rows(idx, weight, dim)
    return out.reshape(b, h, dim)
